# Initial kernel scaffold; baseline (speedup 1.0000x reference)
import jax
import jax.numpy as jnp
from jax.experimental import pallas as pl


def kernel(h, labels, centers):
    raise NotImplementedError("write your pallas kernel here")


# speedup vs baseline: 13.0358x; 13.0358x over previous
import jax
import jax.numpy as jnp
from jax.experimental import pallas as pl
from jax.experimental.pallas import tpu as pltpu

_BATCH_BLOCK = 2048


def _mse_kernel(lab_ref, h_ref, c_ref, out_ref, acc_ref):
    i = pl.program_id(0)
    n = pl.num_programs(0)

    @pl.when(i == 0)
    def _init():
        acc_ref[0] = 0.0

    row = lab_ref[0] % 8
    c = c_ref[pl.ds(row, 1), :]
    diff = h_ref[...] - c
    acc_ref[0] += jnp.sum(diff * diff)

    @pl.when(i == n - 1)
    def _fin():
        out_ref[0, 0] = acc_ref[0]


def kernel(h, labels, centers):
    B, D = h.shape
    lab = jnp.asarray(labels, dtype=jnp.int32).reshape((1,))
    grid = (B // _BATCH_BLOCK,)
    total = pl.pallas_call(
        _mse_kernel,
        grid_spec=pltpu.PrefetchScalarGridSpec(
            num_scalar_prefetch=1,
            grid=grid,
            in_specs=[
                pl.BlockSpec((_BATCH_BLOCK, D), lambda i, lab_ref: (i, 0)),
                pl.BlockSpec((8, D), lambda i, lab_ref: (lab_ref[0] // 8, 0)),
            ],
            out_specs=pl.BlockSpec(
                (1, 1), lambda i, lab_ref: (0, 0), memory_space=pltpu.SMEM
            ),
            scratch_shapes=[pltpu.SMEM((1,), jnp.float32)],
        ),
        out_shape=jax.ShapeDtypeStruct((1, 1), jnp.float32),
    )(lab, h, centers)
    return (total[0, 0] / (B * D)).astype(jnp.float32)


# speedup vs baseline: 13.4165x; 1.0292x over previous
import functools

import jax
import jax.numpy as jnp
from jax.experimental import pallas as pl
from jax.experimental.pallas import tpu as pltpu

_BATCH_BLOCK = 2048


def _mse_kernel(scale, lab_ref, h_ref, c_ref, out_ref, acc_ref):
    i = pl.program_id(0)
    n = pl.num_programs(0)

    @pl.when(i == 0)
    def _init():
        acc_ref[0] = 0.0

    row = lab_ref[0] % 8
    c = c_ref[pl.ds(row, 1), :]
    diff = h_ref[...] - c
    acc_ref[0] += jnp.sum(diff * diff)

    @pl.when(i == n - 1)
    def _fin():
        out_ref[0, 0] = acc_ref[0] * scale


def kernel(h, labels, centers):
    B, D = h.shape
    lab = jnp.asarray(labels, dtype=jnp.int32).reshape((1,))
    grid = (B // _BATCH_BLOCK,)
    total = pl.pallas_call(
        functools.partial(_mse_kernel, 1.0 / (B * D)),
        grid_spec=pltpu.PrefetchScalarGridSpec(
            num_scalar_prefetch=1,
            grid=grid,
            in_specs=[
                pl.BlockSpec((_BATCH_BLOCK, D), lambda i, lab_ref: (i, 0)),
                pl.BlockSpec((8, D), lambda i, lab_ref: (lab_ref[0] // 8, 0)),
            ],
            out_specs=pl.BlockSpec(
                (1, 1), lambda i, lab_ref: (0, 0), memory_space=pltpu.SMEM
            ),
            scratch_shapes=[pltpu.SMEM((1,), jnp.float32)],
        ),
        out_shape=jax.ShapeDtypeStruct((1, 1), jnp.float32),
    )(lab, h, centers)
    return jnp.reshape(total, ())


# speedup vs baseline: 13.8663x; 1.0335x over previous
import functools

import jax
import jax.numpy as jnp
from jax.experimental import pallas as pl
from jax.experimental.pallas import tpu as pltpu

_BATCH_BLOCK = 16384


def _mse_kernel(scale, lab_ref, h_ref, c_ref, out_ref, acc_ref):
    i = pl.program_id(0)
    n = pl.num_programs(0)

    @pl.when(i == 0)
    def _init():
        acc_ref[0] = 0.0

    row = lab_ref[0] % 8
    c = c_ref[pl.ds(row, 1), :]
    diff = h_ref[...] - c
    acc_ref[0] += jnp.sum(diff * diff)

    @pl.when(i == n - 1)
    def _fin():
        out_ref[0, 0] = acc_ref[0] * scale


def kernel(h, labels, centers):
    B, D = h.shape
    lab = jnp.asarray(labels, dtype=jnp.int32).reshape((1,))
    grid = (B // _BATCH_BLOCK,)
    total = pl.pallas_call(
        functools.partial(_mse_kernel, 1.0 / (B * D)),
        grid_spec=pltpu.PrefetchScalarGridSpec(
            num_scalar_prefetch=1,
            grid=grid,
            in_specs=[
                pl.BlockSpec((_BATCH_BLOCK, D), lambda i, lab_ref: (i, 0)),
                pl.BlockSpec((8, D), lambda i, lab_ref: (lab_ref[0] // 8, 0)),
            ],
            out_specs=pl.BlockSpec(
                (1, 1), lambda i, lab_ref: (0, 0), memory_space=pltpu.SMEM
            ),
            scratch_shapes=[pltpu.SMEM((1,), jnp.float32)],
        ),
        out_shape=jax.ShapeDtypeStruct((1, 1), jnp.float32),
    )(lab, h, centers)
    return jnp.reshape(total, ())
